# baseline (speedup 1.0000x reference)
import functools

import jax
import jax.numpy as jnp
from jax.experimental import pallas as pl
from jax.experimental.pallas import tpu as pltpu

_RATIO = 2.0 / 3.0


def _ohem_body(preds_ref, targets_ref, out_ref, losses_ref, mm_ref, *,
               block_rows, k, d):
    i = pl.program_id(0)
    g = pl.num_programs(0)

    p = preds_ref[...]
    t = targets_ref[...]
    log_p = jnp.maximum(jnp.log(p), -100.0)
    log_1mp = jnp.maximum(jnp.log(1.0 - p), -100.0)
    per_elem = t * (log_1mp - log_p) - log_1mp
    chunks = block_rows // d
    x3 = per_elem.reshape(chunks, d, d)
    xt = jnp.transpose(x3, (0, 2, 1))
    row_sums = jnp.sum(xt, axis=1)
    row_sums = jnp.maximum(row_sums, 0.0)
    losses_ref[pl.ds(i * chunks, chunks), :] = row_sums

    rb = jax.lax.bitcast_convert_type(row_sums, jnp.int32)
    bmin, bmax = jnp.min(rb), jnp.max(rb)

    @pl.when(i == 0)
    def _init_mm():
        mm_ref[0] = bmin
        mm_ref[1] = bmax

    @pl.when(i > 0)
    def _acc_mm():
        mm_ref[0] = jnp.minimum(mm_ref[0], bmin)
        mm_ref[1] = jnp.maximum(mm_ref[1], bmax)

    @pl.when(i == g - 1)
    def _select():
        vals = losses_ref[...]
        bits = jax.lax.bitcast_convert_type(vals, jnp.int32)
        lo0 = jnp.full((1, 1), mm_ref[0], jnp.int32)
        hi0 = jnp.full((1, 1), mm_ref[1], jnp.int32)

        def body(_, carry):
            lo, hi = carry
            step = jnp.maximum((hi - lo) // 8, 1)
            m = jnp.zeros((1, 1), jnp.int32)
            for j in range(1, 8):
                cnt = jnp.sum((bits >= lo + j * step).astype(jnp.int32),
                              keepdims=True)
                m = m + (cnt >= k).astype(jnp.int32)
            new_lo = lo + m * step
            new_hi = jnp.where(m == 7, hi, lo + (m + 1) * step - 1)
            return (new_lo, new_hi)

        lo, hi = jax.lax.fori_loop(0, 5, body, (lo0, hi0))

        step = jnp.maximum((hi - lo) // 8, 1)
        cnts, sums = [], []
        for j in range(8):
            ge = bits >= lo + j * step
            cnts.append(jnp.sum(ge.astype(jnp.int32), keepdims=True))
            sums.append(jnp.sum(jnp.where(ge, vals, 0.0), keepdims=True))
        m = jnp.zeros((1, 1), jnp.int32)
        for j in range(1, 8):
            m = m + (cnts[j] >= k).astype(jnp.int32)
        cnt_m = cnts[0]
        sum_m = sums[0]
        for j in range(1, 8):
            sel = m == j
            cnt_m = jnp.where(sel, cnts[j], cnt_m)
            sum_m = jnp.where(sel, sums[j], sum_m)
        t_m = lo + m * step
        v_m = jax.lax.bitcast_convert_type(t_m, jnp.float32)
        total = sum_m - (cnt_m - k).astype(jnp.float32) * v_m
        out_ref[0, 0] = total[0, 0] / (jnp.float32(k) * jnp.float32(d))


@functools.partial(jax.jit, static_argnames=("interpret",))
def kernel(preds, targets, interpret=False):
    n, d = preds.shape
    k = int(_RATIO * n)
    block_rows = 16384
    grid = (n // block_rows,)
    out = pl.pallas_call(
        functools.partial(_ohem_body, block_rows=block_rows, k=k, d=d),
        grid=grid,
        in_specs=[
            pl.BlockSpec((block_rows, d), lambda i: (i, 0)),
            pl.BlockSpec((block_rows, d), lambda i: (i, 0)),
        ],
        out_specs=pl.BlockSpec(memory_space=pltpu.SMEM),
        out_shape=jax.ShapeDtypeStruct((1, 1), jnp.float32),
        scratch_shapes=[
            pltpu.VMEM((n // d, d), jnp.float32),
            pltpu.SMEM((2,), jnp.int32),
        ],
        compiler_params=pltpu.CompilerParams(
            dimension_semantics=("arbitrary",),
        ),
        interpret=interpret,
    )(preds, targets)
    return out[0, 0]


# speedup vs baseline: 1.1847x; 1.1847x over previous
import functools

import jax
import jax.numpy as jnp
from jax.experimental import pallas as pl
from jax.experimental.pallas import tpu as pltpu

_RATIO = 2.0 / 3.0


def _ohem_body(preds_ref, targets_ref, out_ref, losses_ref, mm_ref, *,
               block_rows, k, d):
    i = pl.program_id(0)
    g = pl.num_programs(0)

    p = preds_ref[...]
    t = targets_ref[...]
    log_p = jnp.maximum(jnp.log(p), -100.0)
    log_1mp = jnp.maximum(jnp.log(1.0 - p), -100.0)
    per_elem = t * (log_1mp - log_p) - log_1mp
    chunks = block_rows // d
    x3 = per_elem.reshape(chunks, d, d)
    xt = jnp.transpose(x3, (0, 2, 1))
    row_sums = jnp.sum(xt, axis=1)
    row_sums = jnp.maximum(row_sums, 0.0)
    losses_ref[pl.ds(i * chunks, chunks), :] = row_sums

    rb = jax.lax.bitcast_convert_type(row_sums, jnp.int32)
    bmin, bmax = jnp.min(rb), jnp.max(rb)

    @pl.when(i == 0)
    def _init_mm():
        mm_ref[0] = bmin
        mm_ref[1] = bmax

    @pl.when(i > 0)
    def _acc_mm():
        mm_ref[0] = jnp.minimum(mm_ref[0], bmin)
        mm_ref[1] = jnp.maximum(mm_ref[1], bmax)

    @pl.when(i == g - 1)
    def _select():
        vals = losses_ref[...]
        bits = jax.lax.bitcast_convert_type(vals, jnp.int32)
        lo0 = jnp.full((1, 1), mm_ref[0], jnp.int32)
        hi0 = jnp.full((1, 1), mm_ref[1], jnp.int32)

        def body(_, carry):
            lo, hi = carry
            step = jnp.maximum((hi - lo) // 8, 1)
            parts = [
                jnp.sum((bits >= lo + j * step).astype(jnp.int32), axis=0,
                        keepdims=True)
                for j in range(1, 8)
            ]
            cnts = jnp.sum(jnp.concatenate(parts, axis=0), axis=1,
                           keepdims=True)
            m = jnp.sum((cnts >= k).astype(jnp.int32), axis=0, keepdims=True)
            new_lo = lo + m * step
            new_hi = jnp.where(m == 7, hi, lo + (m + 1) * step - 1)
            return (new_lo, new_hi)

        lo, hi = jax.lax.fori_loop(0, 5, body, (lo0, hi0))

        step = jnp.maximum((hi - lo) // 8, 1)
        parts = []
        for j in range(8):
            ge = bits >= lo + j * step
            parts.append(jnp.sum(ge.astype(jnp.float32), axis=0,
                                 keepdims=True))
            parts.append(jnp.sum(jnp.where(ge, vals, 0.0), axis=0,
                                 keepdims=True))
        red = jnp.sum(jnp.concatenate(parts, axis=0), axis=1,
                      keepdims=True)
        m = jnp.zeros((1, 1), jnp.int32)
        for j in range(1, 8):
            m = m + (red[2 * j:2 * j + 1] >= k).astype(jnp.int32)
        cnt_m = red[0:1]
        sum_m = red[1:2]
        for j in range(1, 8):
            sel = m == j
            cnt_m = jnp.where(sel, red[2 * j:2 * j + 1], cnt_m)
            sum_m = jnp.where(sel, red[2 * j + 1:2 * j + 2], sum_m)
        t_m = lo + m * step
        v_m = jax.lax.bitcast_convert_type(t_m, jnp.float32)
        total = sum_m - (cnt_m - jnp.float32(k)) * v_m
        out_ref[0, 0] = total[0, 0] / (jnp.float32(k) * jnp.float32(d))


@functools.partial(jax.jit, static_argnames=("interpret",))
def kernel(preds, targets, interpret=False):
    n, d = preds.shape
    k = int(_RATIO * n)
    block_rows = 8192
    grid = (n // block_rows,)
    out = pl.pallas_call(
        functools.partial(_ohem_body, block_rows=block_rows, k=k, d=d),
        grid=grid,
        in_specs=[
            pl.BlockSpec((block_rows, d), lambda i: (i, 0)),
            pl.BlockSpec((block_rows, d), lambda i: (i, 0)),
        ],
        out_specs=pl.BlockSpec(memory_space=pltpu.SMEM),
        out_shape=jax.ShapeDtypeStruct((1, 1), jnp.float32),
        scratch_shapes=[
            pltpu.VMEM((n // d, d), jnp.float32),
            pltpu.SMEM((2,), jnp.int32),
        ],
        compiler_params=pltpu.CompilerParams(
            dimension_semantics=("arbitrary",),
        ),
        interpret=interpret,
    )(preds, targets)
    return out[0, 0]
